# baseline (speedup 1.0000x reference)
import functools

import jax
import jax.numpy as jnp
from jax import lax
from jax.experimental import pallas as pl
from jax.experimental.pallas import tpu as pltpu
from jax.experimental.pallas import tpu_sc as plsc

_LANES = 16
_C = 8
_NP0 = 9
_NP1 = 7
_NXBUF = 3
_NPBUF = 2
_NOBUF = 2
_NSUB = 16


def kernel(x, pe):
    N, L, D = x.shape
    assert L == _NSUB * _C * (_NP0 + _NP1)
    xf = x.reshape(N * L, D)
    mesh = plsc.VectorSubcoreMesh(core_axis_name="c", subcore_axis_name="s")

    @functools.partial(
        pl.kernel,
        out_type=jax.ShapeDtypeStruct((N * L, D), x.dtype),
        mesh=mesh,
        scratch_types=(
            [pltpu.VMEM((_C, D), jnp.float32) for _ in range(_NXBUF + _NPBUF + _NOBUF)]
            + [pltpu.SemaphoreType.DMA] * (_NXBUF + _NPBUF + _NOBUF)
        ),
    )
    def sc_add(x_hbm, pe_hbm, o_hbm, *scratch):
        nbuf = _NXBUF + _NPBUF + _NOBUF
        bufs, sems = scratch[:nbuf], scratch[nbuf:]
        xbufs, xsems = bufs[:_NXBUF], sems[:_NXBUF]
        pbufs, psems = bufs[_NXBUF:_NXBUF + _NPBUF], sems[_NXBUF:_NXBUF + _NPBUF]
        obufs, osems = bufs[_NXBUF + _NPBUF:], sems[_NXBUF + _NPBUF:]

        cid = lax.axis_index("c")
        sid = lax.axis_index("s")

        def pipeline(npieces, pe_base):
            chunk_rows = _C
            nchunks = npieces * N

            def x_row(i):
                p, n = i // N, i % N
                return n * L + pe_base + p * chunk_rows

            def x_copy(i):
                b = i % _NXBUF
                return pltpu.make_async_copy(
                    x_hbm.at[pl.ds(x_row(i), chunk_rows), :],
                    xbufs[b].at[pl.ds(0, chunk_rows), :], xsems[b])

            def pe_copy(p):
                b = p % _NPBUF
                return pltpu.make_async_copy(
                    pe_hbm.at[pl.ds(pe_base + p * chunk_rows, chunk_rows), :],
                    pbufs[b].at[pl.ds(0, chunk_rows), :], psems[b])

            def out_copy(i):
                b = i % _NOBUF
                return pltpu.make_async_copy(
                    obufs[b].at[pl.ds(0, chunk_rows), :],
                    o_hbm.at[pl.ds(x_row(i), chunk_rows), :], osems[b])

            pe_copy(0).start()
            for i in range(min(_NXBUF, nchunks)):
                x_copy(i).start()

            for i in range(nchunks):
                p, n = i // N, i % N
                xb = xbufs[i % _NXBUF]
                ob = obufs[i % _NOBUF]
                pb = pbufs[p % _NPBUF]
                if n == 0:
                    pe_copy(p).wait()
                    if p + 1 < npieces:
                        pe_copy(p + 1).start()
                x_copy(i).wait()
                if i >= _NOBUF:
                    out_copy(i - _NOBUF).wait()

                @pl.loop(0, chunk_rows)
                def _(r):
                    @plsc.parallel_loop(0, D, step=_LANES, unroll=8)
                    def _(c):
                        ob[r, pl.ds(c, _LANES)] = (
                            xb[r, pl.ds(c, _LANES)] + pb[r, pl.ds(c, _LANES)]
                        )

                out_copy(i).start()
                if i + _NXBUF < nchunks:
                    x_copy(i + _NXBUF).start()

            for i in range(max(nchunks - _NOBUF, 0), nchunks):
                out_copy(i).wait()

        @pl.when(cid == 0)
        def _():
            pipeline(_NP0, sid * (_NP0 * _C))

        @pl.when(cid == 1)
        def _():
            pipeline(_NP1, _NSUB * _NP0 * _C + sid * (_NP1 * _C))

    return sc_add(xf, pe).reshape(N, L, D)


# speedup vs baseline: 1.0125x; 1.0125x over previous
import functools

import jax
import jax.numpy as jnp
from jax import lax
from jax.experimental import pallas as pl
from jax.experimental.pallas import tpu as pltpu
from jax.experimental.pallas import tpu_sc as plsc

_LANES = 16
_C = 8
_NP0 = 7
_NP1 = 9
_NXBUF = 3
_NPBUF = 2
_NOBUF = 2
_NSUB = 16


def kernel(x, pe):
    N, L, D = x.shape
    assert L == _NSUB * _C * (_NP0 + _NP1)
    xf = x.reshape(N * L, D)
    mesh = plsc.VectorSubcoreMesh(core_axis_name="c", subcore_axis_name="s")

    @functools.partial(
        pl.kernel,
        out_type=jax.ShapeDtypeStruct((N * L, D), x.dtype),
        mesh=mesh,
        scratch_types=(
            [pltpu.VMEM((_C, D), jnp.float32) for _ in range(_NXBUF + _NPBUF + _NOBUF)]
            + [pltpu.SemaphoreType.DMA] * (_NXBUF + _NPBUF + _NOBUF)
        ),
    )
    def sc_add(x_hbm, pe_hbm, o_hbm, *scratch):
        nbuf = _NXBUF + _NPBUF + _NOBUF
        bufs, sems = scratch[:nbuf], scratch[nbuf:]
        xbufs, xsems = bufs[:_NXBUF], sems[:_NXBUF]
        pbufs, psems = bufs[_NXBUF:_NXBUF + _NPBUF], sems[_NXBUF:_NXBUF + _NPBUF]
        obufs, osems = bufs[_NXBUF + _NPBUF:], sems[_NXBUF + _NPBUF:]

        cid = lax.axis_index("c")
        sid = lax.axis_index("s")

        def pipeline(npieces, pe_base):
            chunk_rows = _C
            nchunks = npieces * N

            def x_row(i):
                p, n = i // N, i % N
                return n * L + pe_base + p * chunk_rows

            def x_copy(i):
                b = i % _NXBUF
                return pltpu.make_async_copy(
                    x_hbm.at[pl.ds(x_row(i), chunk_rows), :],
                    xbufs[b].at[pl.ds(0, chunk_rows), :], xsems[b])

            def pe_copy(p):
                b = p % _NPBUF
                return pltpu.make_async_copy(
                    pe_hbm.at[pl.ds(pe_base + p * chunk_rows, chunk_rows), :],
                    pbufs[b].at[pl.ds(0, chunk_rows), :], psems[b])

            def out_copy(i):
                b = i % _NOBUF
                return pltpu.make_async_copy(
                    obufs[b].at[pl.ds(0, chunk_rows), :],
                    o_hbm.at[pl.ds(x_row(i), chunk_rows), :], osems[b])

            pe_copy(0).start()
            for i in range(min(_NXBUF, nchunks)):
                x_copy(i).start()

            for i in range(nchunks):
                p, n = i // N, i % N
                xb = xbufs[i % _NXBUF]
                ob = obufs[i % _NOBUF]
                pb = pbufs[p % _NPBUF]
                if n == 0:
                    pe_copy(p).wait()
                    if p + 1 < npieces:
                        pe_copy(p + 1).start()
                x_copy(i).wait()
                if i >= _NOBUF:
                    out_copy(i - _NOBUF).wait()

                @pl.loop(0, chunk_rows)
                def _(r):
                    @plsc.parallel_loop(0, D, step=_LANES, unroll=8)
                    def _(c):
                        ob[r, pl.ds(c, _LANES)] = (
                            xb[r, pl.ds(c, _LANES)] + pb[r, pl.ds(c, _LANES)]
                        )

                out_copy(i).start()
                if i + _NXBUF < nchunks:
                    x_copy(i + _NXBUF).start()

            for i in range(max(nchunks - _NOBUF, 0), nchunks):
                out_copy(i).wait()

        @pl.when(cid == 0)
        def _():
            pipeline(_NP0, sid * (_NP0 * _C))

        @pl.when(cid == 1)
        def _():
            pipeline(_NP1, _NSUB * _NP0 * _C + sid * (_NP1 * _C))

    return sc_add(xf, pe).reshape(N, L, D)


# speedup vs baseline: 1.1124x; 1.0987x over previous
import functools

import jax
import jax.numpy as jnp
from jax import lax
from jax.experimental import pallas as pl
from jax.experimental.pallas import tpu as pltpu
from jax.experimental.pallas import tpu_sc as plsc

_LANES = 16
_C = 8
_NXBUF = 3
_NPBUF = 2
_NOBUF = 2
_NWORKERS = 32


def kernel(x, pe):
    N, L, D = x.shape
    lpw = L // _NWORKERS
    npieces = lpw // _C
    nchunks = npieces * N
    xf = x.reshape(N * L, D)
    mesh = plsc.VectorSubcoreMesh(core_axis_name="c", subcore_axis_name="s")

    @functools.partial(
        pl.kernel,
        out_type=jax.ShapeDtypeStruct((N * L, D), x.dtype),
        mesh=mesh,
        scratch_types=(
            [pltpu.VMEM((_C, D), jnp.float32) for _ in range(_NXBUF + _NPBUF + _NOBUF)]
            + [pltpu.SemaphoreType.DMA] * (_NXBUF + _NPBUF + _NOBUF)
        ),
    )
    def sc_add(x_hbm, pe_hbm, o_hbm, *scratch):
        nbuf = _NXBUF + _NPBUF + _NOBUF
        bufs, sems = scratch[:nbuf], scratch[nbuf:]
        xbufs, xsems = bufs[:_NXBUF], sems[:_NXBUF]
        pbufs, psems = bufs[_NXBUF:_NXBUF + _NPBUF], sems[_NXBUF:_NXBUF + _NPBUF]
        obufs, osems = bufs[_NXBUF + _NPBUF:], sems[_NXBUF + _NPBUF:]

        wid = lax.axis_index("c") * 16 + lax.axis_index("s")
        pe_base = wid * lpw

        def x_row(i):
            p, n = i // N, i % N
            return n * L + pe_base + p * _C

        def x_copy(i):
            b = i % _NXBUF
            return pltpu.make_async_copy(
                x_hbm.at[pl.ds(x_row(i), _C), :], xbufs[b], xsems[b])

        def pe_copy(p):
            b = p % _NPBUF
            return pltpu.make_async_copy(
                pe_hbm.at[pl.ds(pe_base + p * _C, _C), :], pbufs[b], psems[b])

        def out_copy(i):
            b = i % _NOBUF
            return pltpu.make_async_copy(
                obufs[b], o_hbm.at[pl.ds(x_row(i), _C), :], osems[b])

        pe_copy(0).start()
        for i in range(min(_NXBUF, nchunks)):
            x_copy(i).start()

        for i in range(nchunks):
            p, n = i // N, i % N
            xb = xbufs[i % _NXBUF]
            ob = obufs[i % _NOBUF]
            pb = pbufs[p % _NPBUF]
            if n == 0:
                pe_copy(p).wait()
                if p + 1 < npieces:
                    pe_copy(p + 1).start()
            x_copy(i).wait()
            if i >= _NOBUF:
                out_copy(i - _NOBUF).wait()

            @pl.loop(0, _C)
            def _(r):
                @plsc.parallel_loop(0, D, step=_LANES, unroll=8)
                def _(c):
                    ob[r, pl.ds(c, _LANES)] = (
                        xb[r, pl.ds(c, _LANES)] + pb[r, pl.ds(c, _LANES)]
                    )

            out_copy(i).start()
            if i + _NXBUF < nchunks:
                x_copy(i + _NXBUF).start()

        for i in range(max(nchunks - _NOBUF, 0), nchunks):
            out_copy(i).wait()

    return sc_add(xf, pe).reshape(N, L, D)
